# baseline (speedup 1.0000x reference)
import functools

import jax
import jax.numpy as jnp
from jax import lax
from jax.experimental import pallas as pl
from jax.experimental.pallas import tpu as pltpu
from jax.experimental.pallas import tpu_sc as plsc

NC = 2
NS = 16
NW = NC * NS
B = 128
GRP = 4
GRP1 = 3
PC = 8

f32 = jnp.float32


def _mesh():
    return plsc.VectorSubcoreMesh(
        core_axis_name="c", subcore_axis_name="s", num_cores=NC, num_subcores=NS
    )


def _rsqrt16(v):
    i = lax.bitcast_convert_type(v, jnp.int32)
    i = jnp.int32(0x5F3759DF) - lax.shift_right_logical(i, 1)
    y = lax.bitcast_convert_type(i, f32)
    for _ in range(3):
        y = y * (1.5 - 0.5 * v * y * y)
    return y


def _elu16(v):
    return jnp.where(v > 0.0, v, jnp.exp(jnp.minimum(v, 0.0)) - 1.0)


def _fetch_idx(ei_hbm, row, w, epw, buf, n):
    pltpu.sync_copy(ei_hbm.at[row, pl.ds(w * epw, epw)], buf.at[pl.ds(0, epw)])
    total = buf.shape[0]
    lanes = lax.iota(jnp.int32, 16)
    for i in range((total - epw) // 16):
        buf[pl.ds(epw + i * 16, 16)] = n + lanes + 16 * (i % 7)


def _deg_call(ei, zeros, ones, n, epw, n_pad, ch):
    rps = n_pad // NS

    @functools.partial(
        pl.kernel,
        out_type=jax.ShapeDtypeStruct((NC, n_pad, 16), f32),
        mesh=_mesh(),
        scratch_types=[
            pltpu.VMEM((ch * B,), jnp.int32),
            pltpu.VMEM((B, 16), f32),
            pltpu.VMEM_SHARED((n_pad, 16), f32),
            pltpu.SemaphoreType.DMA,
        ],
        compiler_params=pltpu.CompilerParams(use_tc_tiling_on_sc=False),
    )
    def k(ei_hbm, zeros_hbm, ones_hbm, out_hbm, dst_v, ones_v, acc_sh, dsem):
        c = lax.axis_index("c")
        s = lax.axis_index("s")
        w = c * NS + s
        pltpu.sync_copy(zeros_hbm.at[pl.ds(s * rps, rps)],
                        acc_sh.at[pl.ds(s * rps, rps)])
        pltpu.sync_copy(ones_hbm, ones_v)
        _fetch_idx(ei_hbm, 1, w, epw, dst_v, n)
        plsc.subcore_barrier()

        def body(j, carry):
            pltpu.async_copy(ones_v, acc_sh.at[dst_v.at[pl.ds(j * B, B)]],
                             dsem, add=True)
            return carry

        lax.fori_loop(0, ch, body, 0)

        def drain(j, carry):
            pltpu.make_async_copy(ones_v, acc_sh.at[dst_v.at[pl.ds(0, B)]],
                                  dsem).wait()
            return carry

        lax.fori_loop(0, ch, drain, 0)
        plsc.subcore_barrier()
        pltpu.sync_copy(acc_sh.at[pl.ds(s * rps, rps)],
                        out_hbm.at[c, pl.ds(s * rps, rps)])

    return k(ei, zeros, ones)


def _edge_pipeline(src_v, dst_v, rows_v, acc_sh, tab_sh, gsem, ssem, ch, grp):
    GRP = grp
    ngrp = ch // GRP
    slots = 2 * GRP

    def gissue(j, slot):
        pltpu.async_copy(tab_sh.at[src_v.at[pl.ds(j * B, B)]],
                         rows_v.at[slot], gsem.at[slot])

    def gwait(slot):
        pltpu.make_async_copy(tab_sh.at[src_v.at[pl.ds(0, B)]],
                              rows_v.at[slot], gsem.at[slot]).wait()

    def sissue(j, slot):
        pltpu.async_copy(rows_v.at[slot],
                         acc_sh.at[dst_v.at[pl.ds(j * B, B)]],
                         ssem.at[slot], add=True)

    def swait(slot):
        pltpu.make_async_copy(rows_v.at[slot],
                              acc_sh.at[dst_v.at[pl.ds(0, B)]],
                              ssem.at[slot]).wait()

    for b in range(GRP):
        gissue(b, b)

    def body(g, carry):
        h = g % 2
        base = h * GRP
        ob = (1 - h) * GRP
        for b in range(GRP):
            gwait(base + b)
        for b in range(GRP):
            sissue(g * GRP + b, base + b)

        @pl.when(g + 1 < ngrp)
        def _():
            for b in range(GRP):
                @pl.when(g >= 1)
                def _():
                    swait(ob + b)
                gissue((g + 1) * GRP + b, ob + b)
        return carry

    lax.fori_loop(0, ngrp, body, 0)
    for b in range(slots):
        swait(b)


def _agg1_call(ei, h1, degs, n, epw, n_pad, ch, hid):
    rps = n_pad // NS
    pr = rps // PC
    wf = hid + 16

    @functools.partial(
        pl.kernel,
        out_type=(
            jax.ShapeDtypeStruct((NC, n_pad, wf), f32),
            jax.ShapeDtypeStruct((NC, n_pad, wf), f32),
        ),
        mesh=_mesh(),
        scratch_types=[
            pltpu.VMEM((ch * B,), jnp.int32),
            pltpu.VMEM((ch * B,), jnp.int32),
            pltpu.VMEM((2 * GRP1, B, wf), f32),
            pltpu.VMEM((pr, 16), f32),
            pltpu.VMEM((pr, 16), f32),
            pltpu.VMEM((pr, hid), f32),
            pltpu.VMEM((pr, wf), f32),
            pltpu.VMEM_SHARED((n_pad, wf), f32),
            pltpu.VMEM_SHARED((n_pad, wf), f32),
            pltpu.SemaphoreType.DMA((2 * GRP1,)),
            pltpu.SemaphoreType.DMA((2 * GRP1,)),
        ],
        compiler_params=pltpu.CompilerParams(use_tc_tiling_on_sc=False),
    )
    def k(ei_hbm, h1_hbm, deg_hbm, out_hbm, t1x_hbm, src_v, dst_v, rows_v,
          d0b, d1b, h1b, t1b, acc_sh, tab_sh, gsem, ssem):
        c = lax.axis_index("c")
        s = lax.axis_index("s")
        w = c * NS + s
        _fetch_idx(ei_hbm, 0, w, epw, src_v, n)
        _fetch_idx(ei_hbm, 1, w, epw, dst_v, n)
        for pcc in range(PC):
            r0 = s * rps + pcc * pr
            pltpu.sync_copy(deg_hbm.at[0, pl.ds(r0, pr)], d0b)
            pltpu.sync_copy(deg_hbm.at[1, pl.ds(r0, pr)], d1b)
            pltpu.sync_copy(h1_hbm.at[pl.ds(r0, pr)], h1b)

            def prow(r, carry):
                dv = d0b[r] + d1b[r] + 1.0
                y = _rsqrt16(dv)
                for q in range(hid // 16):
                    t1b[r, pl.ds(q * 16, 16)] = h1b[r, pl.ds(q * 16, 16)] * y
                t1b[r, pl.ds(hid, 16)] = y
                return carry

            lax.fori_loop(0, pr, prow, 0)
            pltpu.sync_copy(t1b, t1x_hbm.at[c, pl.ds(r0, pr)])
        pltpu.sync_copy(t1x_hbm.at[c, pl.ds(s * rps, rps)],
                        tab_sh.at[pl.ds(s * rps, rps)])
        pltpu.sync_copy(t1x_hbm.at[c, pl.ds(s * rps, rps)],
                        acc_sh.at[pl.ds(s * rps, rps)])
        plsc.subcore_barrier()
        _edge_pipeline(src_v, dst_v, rows_v, acc_sh, tab_sh, gsem, ssem,
                       ch, GRP1)
        plsc.subcore_barrier()
        pltpu.sync_copy(acc_sh.at[pl.ds(s * rps, rps)],
                        out_hbm.at[c, pl.ds(s * rps, rps)])

    return k(ei, h1, degs)


def _agg2_call(ei, h1, degs, acc1, zeros_hid, n, epw, n_pad, ch, hid):
    rps = n_pad // NS
    pr = rps // PC
    wf = hid + 16

    @functools.partial(
        pl.kernel,
        out_type=(
            jax.ShapeDtypeStruct((NC, n_pad, hid), f32),
            jax.ShapeDtypeStruct((NC, n_pad, hid), f32),
            jax.ShapeDtypeStruct((n_pad, 16), f32),
            jax.ShapeDtypeStruct((n_pad, 16), f32),
        ),
        mesh=_mesh(),
        scratch_types=[
            pltpu.VMEM((ch * B,), jnp.int32),
            pltpu.VMEM((ch * B,), jnp.int32),
            pltpu.VMEM((2 * GRP, B, hid), f32),
            pltpu.VMEM((pr, 16), f32),
            pltpu.VMEM((pr, 16), f32),
            pltpu.VMEM((pr, hid), f32),
            pltpu.VMEM((pr, wf), f32),
            pltpu.VMEM((pr, wf), f32),
            pltpu.VMEM((pr, hid), f32),
            pltpu.VMEM((pr, 16), f32),
            pltpu.VMEM((pr, 16), f32),
            pltpu.VMEM_SHARED((n_pad, hid), f32),
            pltpu.VMEM_SHARED((n_pad, hid), f32),
            pltpu.SemaphoreType.DMA((2 * GRP,)),
            pltpu.SemaphoreType.DMA((2 * GRP,)),
        ],
        compiler_params=pltpu.CompilerParams(use_tc_tiling_on_sc=False),
    )
    def k(ei_hbm, h1_hbm, deg_hbm, acc1_hbm, zeros_hbm,
          out_hbm, t2x_hbm, s_hbm, n_hbm,
          src_v, dst_v, rows_v, d0b, d1b, h1b, a0b, a1b, t2b, sb, nb,
          acc_sh, tab_sh, gsem, ssem):
        c = lax.axis_index("c")
        s = lax.axis_index("s")
        w = c * NS + s
        _fetch_idx(ei_hbm, 0, w, epw, src_v, n)
        _fetch_idx(ei_hbm, 1, w, epw, dst_v, n)
        for pcc in range(PC):
            r0 = s * rps + pcc * pr
            pltpu.sync_copy(deg_hbm.at[0, pl.ds(r0, pr)], d0b)
            pltpu.sync_copy(deg_hbm.at[1, pl.ds(r0, pr)], d1b)
            pltpu.sync_copy(h1_hbm.at[pl.ds(r0, pr)], h1b)
            pltpu.sync_copy(acc1_hbm.at[0, pl.ds(r0, pr)], a0b)
            pltpu.sync_copy(acc1_hbm.at[1, pl.ds(r0, pr)], a1b)

            def prow(r, carry):
                dv = d0b[r] + d1b[r] + 1.0
                y = _rsqrt16(dv)
                for q in range(hid // 16):
                    col = pl.ds(q * 16, 16)
                    accv = a0b[r, col] + a1b[r, col]
                    t1v = h1b[r, col] * y
                    g = _elu16(y * (accv - t1v))
                    t2b[r, col] = g * y
                zcol = pl.ds(hid, 16)
                zv = a0b[r, zcol] + a1b[r, zcol] - 2.0 * y
                sb[r] = y * zv + y * y
                nb[r] = y
                return carry

            lax.fori_loop(0, pr, prow, 0)
            pltpu.sync_copy(t2b, t2x_hbm.at[c, pl.ds(r0, pr)])

            @pl.when(c == 0)
            def _():
                pltpu.sync_copy(sb, s_hbm.at[pl.ds(r0, pr)])
                pltpu.sync_copy(nb, n_hbm.at[pl.ds(r0, pr)])
        pltpu.sync_copy(t2x_hbm.at[c, pl.ds(s * rps, rps)],
                        tab_sh.at[pl.ds(s * rps, rps)])

        @pl.when(c == 0)
        def _():
            pltpu.sync_copy(t2x_hbm.at[c, pl.ds(s * rps, rps)],
                            acc_sh.at[pl.ds(s * rps, rps)])

        @pl.when(c != 0)
        def _():
            pltpu.sync_copy(zeros_hbm.at[pl.ds(s * rps, rps)],
                            acc_sh.at[pl.ds(s * rps, rps)])
        plsc.subcore_barrier()
        _edge_pipeline(src_v, dst_v, rows_v, acc_sh, tab_sh, gsem, ssem,
                       ch, GRP)
        plsc.subcore_barrier()
        pltpu.sync_copy(acc_sh.at[pl.ds(s * rps, rps)],
                        out_hbm.at[c, pl.ds(s * rps, rps)])

    return k(ei, h1, degs, acc1, zeros_hid)


def _tc_b1_call(x, w1, b1, n, n_pad, hid):
    def body(x_ref, w_ref, b_ref, h1_ref):
        h1_ref[:n] = jnp.dot(x_ref[...], w_ref[...],
                             preferred_element_type=f32) + b_ref[...]
        h1_ref[n:] = jnp.zeros((n_pad - n, hid), f32)

    return pl.pallas_call(
        body,
        out_shape=jax.ShapeDtypeStruct((n_pad, hid), f32),
    )(x, w1, b1)


def _tc_d_call(acc2, norm16, s16, w2, b2, n, f_out):
    def body(acc_ref, norm_ref, s_ref, w_ref, b_ref, out_ref):
        norm = norm_ref[:n, 0:1]
        ag = norm * (acc_ref[0, :n] + acc_ref[1, :n])
        a = (jnp.dot(ag, w_ref[...], preferred_element_type=f32)
             + s_ref[:n, 0:1] * b_ref[...])
        m = jnp.max(a, axis=1, keepdims=True)
        lse = jnp.log(jnp.sum(jnp.exp(a - m), axis=1, keepdims=True))
        out_ref[...] = a - m - lse

    return pl.pallas_call(
        body,
        out_shape=jax.ShapeDtypeStruct((n, f_out), f32),
    )(acc2, norm16, s16, w2, b2)


def kernel(x, edge_index, W1, b1, W2, b2):
    n, f_in = x.shape
    hid = W1.shape[1]
    f_out = W2.shape[1]
    e = edge_index.shape[1]

    align = NS * 8
    n_pad = ((n + 1 + align - 1) // align) * align
    epw = e // NW
    ch = -(-epw // B)
    grps = GRP * GRP1
    ch = max(2 * grps, ((ch + grps - 1) // grps) * grps)

    zeros = jnp.zeros((n_pad, 16), f32)
    zeros_hid = jnp.zeros((n_pad, hid), f32)
    ones = jnp.ones((B, 16), f32)

    h1 = _tc_b1_call(x, W1, b1.reshape(1, hid), n, n_pad, hid)
    degs = _deg_call(edge_index, zeros, ones, n, epw, n_pad, ch)
    acc1, _t1x = _agg1_call(edge_index, h1, degs, n, epw, n_pad, ch, hid)
    acc2, _t2x, s16, norm16 = _agg2_call(edge_index, h1, degs, acc1,
                                         zeros_hid, n, epw, n_pad, ch, hid)
    return _tc_d_call(acc2, norm16, s16, W2, b2.reshape(1, f_out), n, f_out)


# speedup vs baseline: 1.2362x; 1.2362x over previous
import functools

import jax
import jax.numpy as jnp
from jax import lax
from jax.experimental import pallas as pl
from jax.experimental.pallas import tpu as pltpu
from jax.experimental.pallas import tpu_sc as plsc

NC = 2
NS = 16
NW = NC * NS
B = 128
GRP = 4

f32 = jnp.float32


def _mesh():
    return plsc.VectorSubcoreMesh(
        core_axis_name="c", subcore_axis_name="s", num_cores=NC, num_subcores=NS
    )


def _fetch_idx(ei_hbm, row, w, epw, buf, n):
    pltpu.sync_copy(ei_hbm.at[row, pl.ds(w * epw, epw)], buf.at[pl.ds(0, epw)])
    total = buf.shape[0]
    lanes = lax.iota(jnp.int32, 16)
    for i in range((total - epw) // 16):
        buf[pl.ds(epw + i * 16, 16)] = n + lanes + 16 * (i % 7)


def _deg_call(ei, zeros, ones, n, epw, n_pad, ch):
    rps = n_pad // NS

    @functools.partial(
        pl.kernel,
        out_type=jax.ShapeDtypeStruct((NC, n_pad, 16), f32),
        mesh=_mesh(),
        scratch_types=[
            pltpu.VMEM((ch * B,), jnp.int32),
            pltpu.VMEM((B, 16), f32),
            pltpu.VMEM_SHARED((n_pad, 16), f32),
            pltpu.SemaphoreType.DMA,
        ],
        compiler_params=pltpu.CompilerParams(use_tc_tiling_on_sc=False),
    )
    def k(ei_hbm, zeros_hbm, ones_hbm, out_hbm, dst_v, ones_v, acc_sh, dsem):
        c = lax.axis_index("c")
        s = lax.axis_index("s")
        w = c * NS + s
        pltpu.sync_copy(zeros_hbm.at[pl.ds(s * rps, rps)],
                        acc_sh.at[pl.ds(s * rps, rps)])
        pltpu.sync_copy(ones_hbm, ones_v)
        _fetch_idx(ei_hbm, 1, w, epw, dst_v, n)
        plsc.subcore_barrier()

        def body(j, carry):
            pltpu.async_copy(ones_v, acc_sh.at[dst_v.at[pl.ds(j * B, B)]],
                             dsem, add=True)
            return carry

        lax.fori_loop(0, ch, body, 0)

        def drain(j, carry):
            pltpu.make_async_copy(ones_v, acc_sh.at[dst_v.at[pl.ds(0, B)]],
                                  dsem).wait()
            return carry

        lax.fori_loop(0, ch, drain, 0)
        plsc.subcore_barrier()
        pltpu.sync_copy(acc_sh.at[pl.ds(s * rps, rps)],
                        out_hbm.at[c, pl.ds(s * rps, rps)])

    return k(ei, zeros, ones)


def _agg_call(ei, table, n, epw, n_pad, ch, w_feat):
    rps = n_pad // NS
    slots = 2 * GRP
    ngrp = ch // GRP

    @functools.partial(
        pl.kernel,
        out_type=jax.ShapeDtypeStruct((NC, n_pad, w_feat), f32),
        mesh=_mesh(),
        scratch_types=[
            pltpu.VMEM((ch * B,), jnp.int32),
            pltpu.VMEM((ch * B,), jnp.int32),
            pltpu.VMEM((slots, B, w_feat), f32),
            pltpu.VMEM_SHARED((n_pad, w_feat), f32),
            pltpu.VMEM_SHARED((n_pad, w_feat), f32),
            pltpu.SemaphoreType.DMA((slots,)),
            pltpu.SemaphoreType.DMA((slots,)),
        ],
        compiler_params=pltpu.CompilerParams(use_tc_tiling_on_sc=False),
    )
    def k(ei_hbm, table_hbm, out_hbm, src_v, dst_v, rows_v, acc_sh, tab_sh,
          gsem, ssem):
        c = lax.axis_index("c")
        s = lax.axis_index("s")
        w = c * NS + s
        _fetch_idx(ei_hbm, 0, w, epw, src_v, n)
        _fetch_idx(ei_hbm, 1, w, epw, dst_v, n)
        pltpu.sync_copy(table_hbm.at[pl.ds(s * rps, rps)],
                        tab_sh.at[pl.ds(s * rps, rps)])
        pltpu.sync_copy(table_hbm.at[pl.ds(s * rps, rps)],
                        acc_sh.at[pl.ds(s * rps, rps)])
        plsc.subcore_barrier()

        def gissue(j, slot):
            pltpu.async_copy(tab_sh.at[src_v.at[pl.ds(j * B, B)]],
                             rows_v.at[slot], gsem.at[slot])

        def gwait(slot):
            pltpu.make_async_copy(tab_sh.at[src_v.at[pl.ds(0, B)]],
                                  rows_v.at[slot], gsem.at[slot]).wait()

        def sissue(j, slot):
            pltpu.async_copy(rows_v.at[slot],
                             acc_sh.at[dst_v.at[pl.ds(j * B, B)]],
                             ssem.at[slot], add=True)

        def swait(slot):
            pltpu.make_async_copy(rows_v.at[slot],
                                  acc_sh.at[dst_v.at[pl.ds(0, B)]],
                                  ssem.at[slot]).wait()

        for b in range(GRP):
            gissue(b, b)

        def body(g, carry):
            h = g % 2
            base = h * GRP
            ob = (1 - h) * GRP
            for b in range(GRP):
                gwait(base + b)
            for b in range(GRP):
                sissue(g * GRP + b, base + b)

            @pl.when(g + 1 < ngrp)
            def _():
                for b in range(GRP):
                    @pl.when(g >= 1)
                    def _():
                        swait(ob + b)
                    gissue((g + 1) * GRP + b, ob + b)
            return carry

        lax.fori_loop(0, ngrp, body, 0)
        for b in range(slots):
            swait(b)
        plsc.subcore_barrier()
        pltpu.sync_copy(acc_sh.at[pl.ds(s * rps, rps)],
                        out_hbm.at[c, pl.ds(s * rps, rps)])

    return k(ei, table)


def _tc_b1_call(x, w1, b1, n, n_pad, hid):
    def body(x_ref, w_ref, b_ref, h1_ref):
        h1_ref[:n] = jnp.dot(x_ref[...], w_ref[...],
                             preferred_element_type=f32) + b_ref[...]
        h1_ref[n:] = jnp.zeros((n_pad - n, hid), f32)

    return pl.pallas_call(
        body,
        out_shape=jax.ShapeDtypeStruct((n_pad, hid), f32),
    )(x, w1, b1)


def _tc_b2_call(h1, degs, n_pad, hid, blk):
    blk = n_pad

    def body(h1_ref, deg_ref, t1_ref, norm_ref):
        deg = deg_ref[0] + deg_ref[1]
        norm = lax.rsqrt(deg[:, 0:1] + 1.0)
        norm16 = jnp.broadcast_to(norm, (blk, 16))
        t1_ref[...] = jnp.concatenate([h1_ref[...] * norm, norm16], axis=1)
        norm_ref[...] = norm16

    return pl.pallas_call(
        body,
        out_shape=(
            jax.ShapeDtypeStruct((n_pad, hid + 16), f32),
            jax.ShapeDtypeStruct((n_pad, 16), f32),
        ),
    )(h1, degs)


def _tc_c_call(acc1, t1aug, norm16, n_pad, hid, blk):
    blk = n_pad

    def body(acc_ref, t1_ref, norm_ref, t2_ref, s_ref):
        norm = norm_ref[:, 0:1]
        accsum = acc_ref[0] + acc_ref[1]
        agg1 = norm * (accsum[:, :hid] - t1_ref[:, :hid])
        g = jnp.where(agg1 > 0.0,
                      agg1, jnp.exp(jnp.minimum(agg1, 0.0)) - 1.0)
        t2_ref[...] = g * norm
        z = accsum[:, hid:hid + 1] - 2.0 * norm
        s_ref[...] = jnp.broadcast_to(norm * z + norm * norm, (blk, 16))

    return pl.pallas_call(
        body,
        out_shape=(
            jax.ShapeDtypeStruct((n_pad, hid), f32),
            jax.ShapeDtypeStruct((n_pad, 16), f32),
        ),
    )(acc1, t1aug, norm16)


def _tc_d_call(acc2, t2, norm16, s16, w2, b2, n, f_out, hid, blk):
    def body(acc_ref, t2_ref, norm_ref, s_ref, w_ref, b_ref, out_ref):
        norm = norm_ref[:, 0:1]
        ag = norm * (acc_ref[0] + acc_ref[1] - t2_ref[...])
        a = (jnp.dot(ag[:n], w_ref[...], preferred_element_type=f32)
             + s_ref[:n, 0:1] * b_ref[...])
        m = jnp.max(a, axis=1, keepdims=True)
        lse = jnp.log(jnp.sum(jnp.exp(a - m), axis=1, keepdims=True))
        out_ref[...] = a - m - lse

    return pl.pallas_call(
        body,
        out_shape=jax.ShapeDtypeStruct((n, f_out), f32),
    )(acc2, t2, norm16, s16, w2, b2)


def kernel(x, edge_index, W1, b1, W2, b2):
    n, f_in = x.shape
    hid = W1.shape[1]
    f_out = W2.shape[1]
    e = edge_index.shape[1]

    align = NS * 8
    n_pad = ((n + 1 + align - 1) // align) * align
    epw = e // NW
    ch = -(-epw // B)
    ch = max(2 * GRP, ((ch + GRP - 1) // GRP) * GRP)

    zeros = jnp.zeros((n_pad, 16), f32)
    ones = jnp.ones((B, 16), f32)

    h1 = _tc_b1_call(x, W1, b1.reshape(1, hid), n, n_pad, hid)
    degs = _deg_call(edge_index, zeros, ones, n, epw, n_pad, ch)
    t1aug, norm16 = _tc_b2_call(h1, degs, n_pad, hid, n_pad // 16)
    acc1 = _agg_call(edge_index, t1aug, n, epw, n_pad, ch, hid + 16)
    t2, s16 = _tc_c_call(acc1, t1aug, norm16, n_pad, hid, n_pad // 16)
    acc2 = _agg_call(edge_index, t2, n, epw, n_pad, ch, hid)
    return _tc_d_call(acc2, t2, norm16, s16, W2, b2.reshape(1, f_out),
                      n, f_out, hid, n // 10)
